# baseline (device time: 2921678 ns/iter reference)
import jax
import jax.numpy as jnp
from jax import lax
from jax.experimental import pallas as pl
from jax.experimental.pallas import tpu as pltpu

N_DEV = 32


def kernel(x, w_mat):
    m_per, k = x.shape
    _, n_per = w_mat.shape

    def body(x_ref, w_ref, out_ref, comm_ref, send_sems, recv_sems, credit_sem):
        my = lax.axis_index("i")
        left = (my - 1) % N_DEV
        right = (my + 1) % N_DEV

        barrier = pltpu.get_barrier_semaphore()
        for nbr in (left, right):
            pl.semaphore_signal(
                barrier, inc=1, device_id=(nbr,),
                device_id_type=pl.DeviceIdType.MESH,
            )
        pl.semaphore_wait(barrier, 2)

        comm_ref[0] = x_ref[...]

        for h in range(N_DEV - 1):
            send_slot = h % 2
            recv_slot = (h + 1) % 2
            if h >= 1:
                pl.semaphore_wait(credit_sem, 1)
            rdma = pltpu.make_async_remote_copy(
                src_ref=comm_ref.at[send_slot],
                dst_ref=comm_ref.at[recv_slot],
                send_sem=send_sems.at[send_slot],
                recv_sem=recv_sems.at[recv_slot],
                device_id=(right,),
                device_id_type=pl.DeviceIdType.MESH,
            )
            rdma.start()
            origin = (my - h) % N_DEV
            out_ref[pl.ds(origin * m_per, m_per), :] = jnp.dot(
                comm_ref[send_slot], w_ref[...],
                preferred_element_type=jnp.float32,
            )
            rdma.wait_send()
            if h < N_DEV - 2:
                pl.semaphore_signal(
                    credit_sem, inc=1, device_id=(left,),
                    device_id_type=pl.DeviceIdType.MESH,
                )
            rdma.wait_recv()

        origin = (my - (N_DEV - 1)) % N_DEV
        out_ref[pl.ds(origin * m_per, m_per), :] = jnp.dot(
            comm_ref[(N_DEV - 1) % 2], w_ref[...],
            preferred_element_type=jnp.float32,
        )

    return pl.pallas_call(
        body,
        out_shape=jax.ShapeDtypeStruct((N_DEV * m_per, n_per), jnp.float32),
        in_specs=[
            pl.BlockSpec(memory_space=pltpu.VMEM),
            pl.BlockSpec(memory_space=pltpu.VMEM),
        ],
        out_specs=pl.BlockSpec(memory_space=pltpu.VMEM),
        scratch_shapes=[
            pltpu.VMEM((2, m_per, k), x.dtype),
            pltpu.SemaphoreType.DMA((2,)),
            pltpu.SemaphoreType.DMA((2,)),
            pltpu.SemaphoreType.REGULAR,
        ],
        compiler_params=pltpu.CompilerParams(collective_id=0),
    )(x, w_mat)


# device time: 1527251 ns/iter; 1.9130x vs baseline; 1.9130x over previous
import jax
import jax.numpy as jnp
from jax import lax
from jax.experimental import pallas as pl
from jax.experimental.pallas import tpu as pltpu

N_DEV = 32


def kernel(x, w_mat):
    x = x.astype(jnp.bfloat16)
    w_mat = w_mat.astype(jnp.bfloat16)
    m_per, k = x.shape
    _, n_per = w_mat.shape

    def body(x_ref, w_ref, out_ref, comm_ref, send_sems, recv_sems, credit_sem):
        my = lax.axis_index("i")
        left = (my - 1) % N_DEV
        right = (my + 1) % N_DEV

        barrier = pltpu.get_barrier_semaphore()
        for nbr in (left, right):
            pl.semaphore_signal(
                barrier, inc=1, device_id=(nbr,),
                device_id_type=pl.DeviceIdType.MESH,
            )
        pl.semaphore_wait(barrier, 2)

        comm_ref[0] = x_ref[...]

        for h in range(N_DEV - 1):
            send_slot = h % 2
            recv_slot = (h + 1) % 2
            if h >= 1:
                pl.semaphore_wait(credit_sem, 1)
            rdma = pltpu.make_async_remote_copy(
                src_ref=comm_ref.at[send_slot],
                dst_ref=comm_ref.at[recv_slot],
                send_sem=send_sems.at[send_slot],
                recv_sem=recv_sems.at[recv_slot],
                device_id=(right,),
                device_id_type=pl.DeviceIdType.MESH,
            )
            rdma.start()
            origin = (my - h) % N_DEV
            out_ref[pl.ds(origin * m_per, m_per), :] = jnp.dot(
                comm_ref[send_slot], w_ref[...],
                preferred_element_type=jnp.float32,
            )
            rdma.wait_send()
            if h < N_DEV - 2:
                pl.semaphore_signal(
                    credit_sem, inc=1, device_id=(left,),
                    device_id_type=pl.DeviceIdType.MESH,
                )
            rdma.wait_recv()

        origin = (my - (N_DEV - 1)) % N_DEV
        out_ref[pl.ds(origin * m_per, m_per), :] = jnp.dot(
            comm_ref[(N_DEV - 1) % 2], w_ref[...],
            preferred_element_type=jnp.float32,
        )

    return pl.pallas_call(
        body,
        out_shape=jax.ShapeDtypeStruct((N_DEV * m_per, n_per), jnp.float32),
        in_specs=[
            pl.BlockSpec(memory_space=pltpu.VMEM),
            pl.BlockSpec(memory_space=pltpu.VMEM),
        ],
        out_specs=pl.BlockSpec(memory_space=pltpu.VMEM),
        scratch_shapes=[
            pltpu.VMEM((2, m_per, k), x.dtype),
            pltpu.SemaphoreType.DMA((2,)),
            pltpu.SemaphoreType.DMA((2,)),
            pltpu.SemaphoreType.REGULAR,
        ],
        compiler_params=pltpu.CompilerParams(collective_id=0),
    )(x, w_mat)


# device time: 1438712 ns/iter; 2.0308x vs baseline; 1.0615x over previous
import jax
import jax.numpy as jnp
from jax import lax
from jax.experimental import pallas as pl
from jax.experimental.pallas import tpu as pltpu

N_DEV = 32
H_R = N_DEV // 2
H_L = N_DEV - 1 - H_R


def kernel(x, w_mat):
    x = x.astype(jnp.bfloat16)
    w_mat = w_mat.astype(jnp.bfloat16)
    m_per, k = x.shape
    _, n_per = w_mat.shape

    def body(x_ref, w_ref, out_ref, rbuf, lbuf,
             r_send_sems, r_recv_sems, l_send_sems, l_recv_sems,
             credit_r, credit_l):
        my = lax.axis_index("i")
        left = (my - 1) % N_DEV
        right = (my + 1) % N_DEV

        barrier = pltpu.get_barrier_semaphore()
        for nbr in (left, right):
            pl.semaphore_signal(
                barrier, inc=1, device_id=(nbr,),
                device_id_type=pl.DeviceIdType.MESH,
            )
        pl.semaphore_wait(barrier, 2)

        rbuf[0] = x_ref[...]
        lbuf[0] = x_ref[...]

        def gemm(buf, slot, origin):
            out_ref[pl.ds(origin * m_per, m_per), :] = jnp.dot(
                buf[slot], w_ref[...], preferred_element_type=jnp.float32)

        for h in range(H_R):
            s, r = h % 2, (h + 1) % 2
            l_active = h < H_L
            if h >= 1:
                pl.semaphore_wait(credit_r, 1)
                if l_active:
                    pl.semaphore_wait(credit_l, 1)
            r_rdma = pltpu.make_async_remote_copy(
                src_ref=rbuf.at[s], dst_ref=rbuf.at[r],
                send_sem=r_send_sems.at[s], recv_sem=r_recv_sems.at[r],
                device_id=(right,), device_id_type=pl.DeviceIdType.MESH)
            r_rdma.start()
            if l_active:
                l_rdma = pltpu.make_async_remote_copy(
                    src_ref=lbuf.at[s], dst_ref=lbuf.at[r],
                    send_sem=l_send_sems.at[s], recv_sem=l_recv_sems.at[r],
                    device_id=(left,), device_id_type=pl.DeviceIdType.MESH)
                l_rdma.start()
            if h == 0:
                gemm(rbuf, 0, my)
            else:
                gemm(rbuf, s, (my - h) % N_DEV)
                if l_active:
                    gemm(lbuf, s, (my + h) % N_DEV)
            r_rdma.wait_send()
            if h < H_R - 1:
                pl.semaphore_signal(
                    credit_r, inc=1, device_id=(left,),
                    device_id_type=pl.DeviceIdType.MESH)
            if l_active:
                l_rdma.wait_send()
                if h < H_L - 1:
                    pl.semaphore_signal(
                        credit_l, inc=1, device_id=(right,),
                        device_id_type=pl.DeviceIdType.MESH)
            r_rdma.wait_recv()
            if l_active:
                l_rdma.wait_recv()

        gemm(rbuf, H_R % 2, (my - H_R) % N_DEV)
        gemm(lbuf, H_L % 2, (my + H_L) % N_DEV)

    return pl.pallas_call(
        body,
        out_shape=jax.ShapeDtypeStruct((N_DEV * m_per, n_per), jnp.float32),
        in_specs=[
            pl.BlockSpec(memory_space=pltpu.VMEM),
            pl.BlockSpec(memory_space=pltpu.VMEM),
        ],
        out_specs=pl.BlockSpec(memory_space=pltpu.VMEM),
        scratch_shapes=[
            pltpu.VMEM((2, m_per, k), jnp.bfloat16),
            pltpu.VMEM((2, m_per, k), jnp.bfloat16),
            pltpu.SemaphoreType.DMA((2,)),
            pltpu.SemaphoreType.DMA((2,)),
            pltpu.SemaphoreType.DMA((2,)),
            pltpu.SemaphoreType.DMA((2,)),
            pltpu.SemaphoreType.REGULAR,
            pltpu.SemaphoreType.REGULAR,
        ],
        compiler_params=pltpu.CompilerParams(collective_id=0),
    )(x, w_mat)


# device time: 763997 ns/iter; 3.8242x vs baseline; 1.8831x over previous
import numpy as np

import jax
import jax.numpy as jnp
from jax import lax
from jax.experimental import pallas as pl
from jax.experimental.pallas import tpu as pltpu

N_DEV = 32
H_R = N_DEV // 2
H_L = N_DEV - 1 - H_R


def _build_tables():
    harness = []
    for z in range(4):
        for yi in range(4):
            row = [(0, yi, z), (1, yi, z)]
            if yi % 2:
                row = row[::-1]
            harness.extend(row)
    path = []
    for yi in range(4):
        zs = range(4) if yi % 2 == 0 else range(3, -1, -1)
        path.extend((yi, z) for z in zs)
    cyc = [(0, y, z) for (y, z) in path]
    cyc += [(1, y, z) for (y, z) in reversed(path)]
    for a, b in zip(cyc, cyc[1:] + cyc[:1]):
        assert sum(abs(u - v) for u, v in zip(a, b)) == 1, (a, b)
    h_of = {c: i for i, c in enumerate(harness)}
    c2h = np.array([h_of[c] for c in cyc], np.int32)
    pos = np.zeros(N_DEV, np.int32)
    for cp, hidx in enumerate(c2h):
        pos[hidx] = cp
    return pos, c2h


_POS, _C2H = _build_tables()


def kernel(x, w_mat):
    x = x.astype(jnp.bfloat16)
    w_mat = w_mat.astype(jnp.bfloat16)
    m_per, k = x.shape
    _, n_per = w_mat.shape

    def body(pos_ref, c2h_ref, x_ref, w_ref, out_ref, rbuf, lbuf,
             r_send_sems, r_recv_sems, l_send_sems, l_recv_sems,
             credit_r, credit_l):
        my = lax.axis_index("i")
        p = pos_ref[my]
        right = c2h_ref[(p + 1) % N_DEV]
        left = c2h_ref[(p + N_DEV - 1) % N_DEV]

        barrier = pltpu.get_barrier_semaphore()
        for nbr in (left, right):
            pl.semaphore_signal(
                barrier, inc=1, device_id=(nbr,),
                device_id_type=pl.DeviceIdType.MESH,
            )
        pl.semaphore_wait(barrier, 2)

        rbuf[0] = x_ref[...]
        lbuf[0] = x_ref[...]

        def gemm(buf, slot, origin):
            out_ref[pl.ds(origin * m_per, m_per), :] = jnp.dot(
                buf[slot], w_ref[...], preferred_element_type=jnp.float32)

        for h in range(H_R):
            s, r = h % 2, (h + 1) % 2
            l_active = h < H_L
            if h >= 1:
                pl.semaphore_wait(credit_r, 1)
                if l_active:
                    pl.semaphore_wait(credit_l, 1)
            r_rdma = pltpu.make_async_remote_copy(
                src_ref=rbuf.at[s], dst_ref=rbuf.at[r],
                send_sem=r_send_sems.at[s], recv_sem=r_recv_sems.at[r],
                device_id=(right,), device_id_type=pl.DeviceIdType.MESH)
            r_rdma.start()
            if l_active:
                l_rdma = pltpu.make_async_remote_copy(
                    src_ref=lbuf.at[s], dst_ref=lbuf.at[r],
                    send_sem=l_send_sems.at[s], recv_sem=l_recv_sems.at[r],
                    device_id=(left,), device_id_type=pl.DeviceIdType.MESH)
                l_rdma.start()
            if h == 0:
                gemm(rbuf, 0, my)
            else:
                gemm(rbuf, s, c2h_ref[(p + N_DEV - h) % N_DEV])
                if l_active:
                    gemm(lbuf, s, c2h_ref[(p + h) % N_DEV])
            r_rdma.wait_send()
            if h < H_R - 1:
                pl.semaphore_signal(
                    credit_r, inc=1, device_id=(left,),
                    device_id_type=pl.DeviceIdType.MESH)
            if l_active:
                l_rdma.wait_send()
                if h < H_L - 1:
                    pl.semaphore_signal(
                        credit_l, inc=1, device_id=(right,),
                        device_id_type=pl.DeviceIdType.MESH)
            r_rdma.wait_recv()
            if l_active:
                l_rdma.wait_recv()

        gemm(rbuf, H_R % 2, c2h_ref[(p + N_DEV - H_R) % N_DEV])
        gemm(lbuf, H_L % 2, c2h_ref[(p + H_L) % N_DEV])

    return pl.pallas_call(
        body,
        out_shape=jax.ShapeDtypeStruct((N_DEV * m_per, n_per), jnp.float32),
        in_specs=[
            pl.BlockSpec(memory_space=pltpu.SMEM),
            pl.BlockSpec(memory_space=pltpu.SMEM),
            pl.BlockSpec(memory_space=pltpu.VMEM),
            pl.BlockSpec(memory_space=pltpu.VMEM),
        ],
        out_specs=pl.BlockSpec(memory_space=pltpu.VMEM),
        scratch_shapes=[
            pltpu.VMEM((2, m_per, k), jnp.bfloat16),
            pltpu.VMEM((2, m_per, k), jnp.bfloat16),
            pltpu.SemaphoreType.DMA((2,)),
            pltpu.SemaphoreType.DMA((2,)),
            pltpu.SemaphoreType.DMA((2,)),
            pltpu.SemaphoreType.DMA((2,)),
            pltpu.SemaphoreType.REGULAR,
            pltpu.SemaphoreType.REGULAR,
        ],
        compiler_params=pltpu.CompilerParams(collective_id=0),
    )(jnp.asarray(_POS), jnp.asarray(_C2H), x, w_mat)


# device time: 748484 ns/iter; 3.9035x vs baseline; 1.0207x over previous
import numpy as np

import jax
import jax.numpy as jnp
from jax import lax
from jax.experimental import pallas as pl
from jax.experimental.pallas import tpu as pltpu

N_DEV = 32
H_R = N_DEV // 2
H_L = N_DEV - 1 - H_R


def _build_tables():
    harness = []
    for z in range(4):
        for yi in range(4):
            row = [(0, yi, z), (1, yi, z)]
            if yi % 2:
                row = row[::-1]
            harness.extend(row)
    path = []
    for yi in range(4):
        zs = range(4) if yi % 2 == 0 else range(3, -1, -1)
        path.extend((yi, z) for z in zs)
    cyc = [(0, y, z) for (y, z) in path]
    cyc += [(1, y, z) for (y, z) in reversed(path)]
    for a, b in zip(cyc, cyc[1:] + cyc[:1]):
        assert sum(abs(u - v) for u, v in zip(a, b)) == 1, (a, b)
    h_of = {c: i for i, c in enumerate(harness)}
    c2h = np.array([h_of[c] for c in cyc], np.int32)
    pos = np.zeros(N_DEV, np.int32)
    for cp, hidx in enumerate(c2h):
        pos[hidx] = cp
    return pos, c2h


_POS, _C2H = _build_tables()


def kernel(x, w_mat):
    x = x.astype(jnp.bfloat16)
    w_mat = w_mat.astype(jnp.bfloat16)
    m_per, k = x.shape
    _, n_per = w_mat.shape

    def body(pos_ref, c2h_ref, x_ref, w_ref, out_ref, rbuf, lbuf,
             half_sr, half_sl, half_rr, half_rl,
             r_send_sems, r_recv_sems, l_send_sems, l_recv_sems,
             credit_r, credit_l):
        my = lax.axis_index("i")
        p = pos_ref[my]
        right = c2h_ref[(p + 1) % N_DEV]
        left = c2h_ref[(p + N_DEV - 1) % N_DEV]

        barrier = pltpu.get_barrier_semaphore()
        for nbr in (left, right):
            pl.semaphore_signal(
                barrier, inc=1, device_id=(nbr,),
                device_id_type=pl.DeviceIdType.MESH,
            )
        pl.semaphore_wait(barrier, 2)

        rbuf[0] = x_ref[...]
        lbuf[0] = x_ref[...]

        def gemm(buf, slot, origin):
            out_ref[pl.ds(origin * m_per, m_per), :] = jnp.dot(
                buf[slot], w_ref[...], preferred_element_type=jnp.float32)

        m_half = m_per // 2
        for h in range(H_R):
            s, r = h % 2, (h + 1) % 2
            last = h == H_R - 1
            if h >= 1:
                pl.semaphore_wait(credit_r, 1)
                pl.semaphore_wait(credit_l, 1)
            if last:
                half_sr[...] = rbuf[s, :m_half, :]
                half_sl[...] = lbuf[s, m_half:, :]
                r_rdma = pltpu.make_async_remote_copy(
                    src_ref=half_sr, dst_ref=half_rr,
                    send_sem=r_send_sems.at[s], recv_sem=r_recv_sems.at[r],
                    device_id=(right,), device_id_type=pl.DeviceIdType.MESH)
                l_rdma = pltpu.make_async_remote_copy(
                    src_ref=half_sl, dst_ref=half_rl,
                    send_sem=l_send_sems.at[s], recv_sem=l_recv_sems.at[r],
                    device_id=(left,), device_id_type=pl.DeviceIdType.MESH)
            else:
                r_rdma = pltpu.make_async_remote_copy(
                    src_ref=rbuf.at[s], dst_ref=rbuf.at[r],
                    send_sem=r_send_sems.at[s], recv_sem=r_recv_sems.at[r],
                    device_id=(right,), device_id_type=pl.DeviceIdType.MESH)
                l_rdma = pltpu.make_async_remote_copy(
                    src_ref=lbuf.at[s], dst_ref=lbuf.at[r],
                    send_sem=l_send_sems.at[s], recv_sem=l_recv_sems.at[r],
                    device_id=(left,), device_id_type=pl.DeviceIdType.MESH)
            r_rdma.start()
            l_rdma.start()
            if h == 0:
                gemm(rbuf, 0, my)
            else:
                gemm(rbuf, s, c2h_ref[(p + N_DEV - h) % N_DEV])
                gemm(lbuf, s, c2h_ref[(p + h) % N_DEV])
            r_rdma.wait_send()
            l_rdma.wait_send()
            if not last:
                pl.semaphore_signal(
                    credit_r, inc=1, device_id=(left,),
                    device_id_type=pl.DeviceIdType.MESH)
                pl.semaphore_signal(
                    credit_l, inc=1, device_id=(right,),
                    device_id_type=pl.DeviceIdType.MESH)
            r_rdma.wait_recv()
            l_rdma.wait_recv()

        anti = c2h_ref[(p + H_R) % N_DEV]
        out_ref[pl.ds(anti * m_per, m_half), :] = jnp.dot(
            half_rr[...], w_ref[...], preferred_element_type=jnp.float32)
        out_ref[pl.ds(anti * m_per + m_half, m_half), :] = jnp.dot(
            half_rl[...], w_ref[...], preferred_element_type=jnp.float32)

    return pl.pallas_call(
        body,
        out_shape=jax.ShapeDtypeStruct((N_DEV * m_per, n_per), jnp.float32),
        in_specs=[
            pl.BlockSpec(memory_space=pltpu.SMEM),
            pl.BlockSpec(memory_space=pltpu.SMEM),
            pl.BlockSpec(memory_space=pltpu.VMEM),
            pl.BlockSpec(memory_space=pltpu.VMEM),
        ],
        out_specs=pl.BlockSpec(memory_space=pltpu.VMEM),
        scratch_shapes=[
            pltpu.VMEM((2, m_per, k), jnp.bfloat16),
            pltpu.VMEM((2, m_per, k), jnp.bfloat16),
            pltpu.VMEM((m_per // 2, k), jnp.bfloat16),
            pltpu.VMEM((m_per // 2, k), jnp.bfloat16),
            pltpu.VMEM((m_per // 2, k), jnp.bfloat16),
            pltpu.VMEM((m_per // 2, k), jnp.bfloat16),
            pltpu.SemaphoreType.DMA((2,)),
            pltpu.SemaphoreType.DMA((2,)),
            pltpu.SemaphoreType.DMA((2,)),
            pltpu.SemaphoreType.DMA((2,)),
            pltpu.SemaphoreType.REGULAR,
            pltpu.SemaphoreType.REGULAR,
        ],
        compiler_params=pltpu.CompilerParams(
            collective_id=0, vmem_limit_bytes=48 * 1024 * 1024),
    )(jnp.asarray(_POS), jnp.asarray(_C2H), x, w_mat)
